# baseline (speedup 1.0000x reference)
import functools

import jax
import jax.numpy as jnp
from jax.experimental import pallas as pl
from jax.experimental.pallas import tpu as pltpu
from jax.experimental.pallas import tpu_sc as plsc

_B = 16384
_D = 64
_GW = 128


def _sc_gather(uid2d, iid2d, user_table, item_table):
    mesh = plsc.VectorSubcoreMesh(core_axis_name="core",
                                  subcore_axis_name="subcore")

    @functools.partial(
        pl.kernel,
        out_type=[jax.ShapeDtypeStruct((_B, _D), jnp.float32),
                  jax.ShapeDtypeStruct((_B, _D), jnp.float32)],
        mesh=mesh,
        compiler_params=pltpu.CompilerParams(use_tc_tiling_on_sc=False),
    )
    def k(uid_hbm, iid_hbm, ut_hbm, it_hbm, u_hbm, v_hbm):
        def body(ui_vmem, ii_vmem, u_vmem, v_vmem):
            pltpu.sync_copy(ut_hbm.at[ui_vmem.at[0]], u_vmem)
            pltpu.sync_copy(it_hbm.at[ii_vmem.at[0]], v_vmem)

        pltpu.emit_pipeline(
            body,
            grid=(_B // _GW,),
            in_specs=[pl.BlockSpec((1, _GW), lambda i: (0, i)),
                      pl.BlockSpec((1, _GW), lambda i: (0, i))],
            out_specs=[pl.BlockSpec((_GW, _D), lambda i: (i, 0)),
                       pl.BlockSpec((_GW, _D), lambda i: (i, 0))],
            core_axis_name=("core", "subcore"),
            dimension_semantics=(pltpu.PARALLEL,),
        )(uid_hbm, iid_hbm, u_hbm, v_hbm)

    return k(uid2d, iid2d, user_table, item_table)


def _bn(x):
    mean = jnp.mean(x, axis=0, keepdims=True)
    var = jnp.mean((x - mean) ** 2, axis=0, keepdims=True)
    return (x - mean) * jax.lax.rsqrt(var + 1e-5)


def _mlp_body(u_ref, v_ref, w1a_ref, w1b_ref, b1_ref, w2_ref, b2_ref,
              w3_ref, b3_ref, w4_ref, b4_ref, out_ref):
    hp = jnp.float32
    prec = jax.lax.Precision.HIGHEST
    h = (jnp.dot(u_ref[:], w1a_ref[:], preferred_element_type=hp, precision=prec)
         + jnp.dot(v_ref[:], w1b_ref[:], preferred_element_type=hp, precision=prec)
         + b1_ref[:])
    h = jnp.maximum(_bn(h), 0.0)
    h = jnp.dot(h, w2_ref[:], preferred_element_type=hp, precision=prec) + b2_ref[:]
    h = jnp.maximum(_bn(h), 0.0)
    h = jnp.dot(h, w3_ref[:], preferred_element_type=hp, precision=prec) + b3_ref[:]
    h = jnp.maximum(_bn(h), 0.0)
    z = jnp.dot(h, w4_ref[:], preferred_element_type=hp, precision=prec) + b4_ref[:]
    out_ref[:] = jax.nn.sigmoid(z) * 5.0 + 1.0


def _tc_mlp(u, v, W1a, W1b, b1, W2, b2, W3, b3, W4, b4):
    return pl.pallas_call(
        _mlp_body,
        out_shape=jax.ShapeDtypeStruct((_B, 1), jnp.float32),
        compiler_params=pltpu.CompilerParams(vmem_limit_bytes=67108864),
    )(u, v, W1a, W1b, b1, W2, b2, W3, b3, W4, b4)


def kernel(user_id, item_id, user_table, item_table,
           W1, b1, W2, b2, W3, b3, W4, b4):
    uid2d = user_id.astype(jnp.int32).reshape(1, _B)
    iid2d = item_id.astype(jnp.int32).reshape(1, _B)
    u, v = _sc_gather(uid2d, iid2d, user_table, item_table)
    return _tc_mlp(u, v, W1[:_D], W1[_D:], b1.reshape(1, -1),
                   W2, b2.reshape(1, -1), W3, b3.reshape(1, -1),
                   W4, b4.reshape(1, -1))


# speedup vs baseline: 1.7500x; 1.7500x over previous
import functools

import jax
import jax.numpy as jnp
from jax.experimental import pallas as pl
from jax.experimental.pallas import tpu as pltpu
from jax.experimental.pallas import tpu_sc as plsc

_B = 16384
_D = 64
_GW = 128


_NW = 32
_RPW = _B // _NW
_CH = 128
_NCH = _RPW // _CH


def _sc_gather(uid, iid, user_table, item_table):
    mesh = plsc.VectorSubcoreMesh(core_axis_name="core",
                                  subcore_axis_name="subcore")

    @functools.partial(
        pl.kernel,
        out_type=[pltpu.HBM((_B, _D), jnp.float32),
                  pltpu.HBM((_B, _D), jnp.float32)],
        mesh=mesh,
        scratch_types=[
            pltpu.VMEM((_RPW,), jnp.int32),
            pltpu.VMEM((_RPW,), jnp.int32),
            pltpu.VMEM((_CH, _D), jnp.float32),
            pltpu.VMEM((_CH, _D), jnp.float32),
            pltpu.SemaphoreType.DMA,
            pltpu.SemaphoreType.DMA,
        ],
    )
    def k(uid_hbm, iid_hbm, ut_hbm, it_hbm, u_hbm, v_hbm,
          uidx_s, iidx_s, ubuf, vbuf, usem, vsem):
        wid = jax.lax.axis_index("subcore") * 2 + jax.lax.axis_index("core")
        base = wid * _RPW
        pltpu.sync_copy(uid_hbm.at[pl.ds(base, _RPW)], uidx_s)
        pltpu.sync_copy(iid_hbm.at[pl.ds(base, _RPW)], iidx_s)

        @pl.loop(0, _NCH)
        def _(g):
            c = g * _CH

            @pl.loop(0, _CH // 16)
            def _(t):
                uvec = uidx_s[pl.ds(c + t * 16, 16)]
                ivec = iidx_s[pl.ds(c + t * 16, 16)]
                for j in range(16):
                    pltpu.async_copy(ut_hbm.at[pl.ds(uvec[j], 1)],
                                     ubuf.at[pl.ds(t * 16 + j, 1)], usem)
                    pltpu.async_copy(it_hbm.at[pl.ds(ivec[j], 1)],
                                     vbuf.at[pl.ds(t * 16 + j, 1)], vsem)

            pltpu.make_async_copy(ut_hbm.at[pl.ds(0, _CH)], ubuf, usem).wait()
            pltpu.make_async_copy(it_hbm.at[pl.ds(0, _CH)], vbuf, vsem).wait()
            pltpu.sync_copy(ubuf, u_hbm.at[pl.ds(base + c, _CH)])
            pltpu.sync_copy(vbuf, v_hbm.at[pl.ds(base + c, _CH)])

    return k(uid, iid, user_table, item_table)


def _bn_relu(h):
    one = jnp.ones((1, _B), jnp.float32)
    s = jnp.dot(one, h, preferred_element_type=jnp.float32)
    q = jnp.dot(one, h * h, preferred_element_type=jnp.float32)
    mean = s * (1.0 / _B)
    var = q * (1.0 / _B) - mean * mean
    a = jax.lax.rsqrt(var + 1e-5)
    return jnp.maximum(h * a - mean * a, 0.0)


def _mlp_body(u_ref, v_ref, w1a_ref, w1b_ref, b1_ref, w2_ref, b2_ref,
              w3_ref, b3_ref, w4_ref, b4_ref, out_ref):
    hp = jnp.float32
    prec = jax.lax.Precision.DEFAULT
    h = (jnp.dot(u_ref[:], w1a_ref[:], preferred_element_type=hp, precision=prec)
         + jnp.dot(v_ref[:], w1b_ref[:], preferred_element_type=hp, precision=prec)
         + b1_ref[:])
    h = _bn_relu(h)
    h = jnp.dot(h, w2_ref[:], preferred_element_type=hp, precision=prec) + b2_ref[:]
    h = _bn_relu(h)
    h = jnp.dot(h, w3_ref[:], preferred_element_type=hp, precision=prec) + b3_ref[:]
    h = _bn_relu(h)
    z = jnp.dot(h, w4_ref[:], preferred_element_type=hp, precision=prec) + b4_ref[:]
    out_ref[:] = jax.nn.sigmoid(z) * 5.0 + 1.0


def _tc_mlp(u, v, W1a, W1b, b1, W2, b2, W3, b3, W4, b4):
    return pl.pallas_call(
        _mlp_body,
        out_shape=jax.ShapeDtypeStruct((_B, 1), jnp.float32),
        compiler_params=pltpu.CompilerParams(vmem_limit_bytes=67108864),
    )(u, v, W1a, W1b, b1, W2, b2, W3, b3, W4, b4)


def kernel(user_id, item_id, user_table, item_table,
           W1, b1, W2, b2, W3, b3, W4, b4):
    u, v = _sc_gather(user_id, item_id, user_table, item_table)
    return _tc_mlp(u, v, W1[:_D], W1[_D:], b1.reshape(1, -1),
                   W2, b2.reshape(1, -1), W3, b3.reshape(1, -1),
                   W4, b4.reshape(1, -1))
